# Initial kernel scaffold; baseline (speedup 1.0000x reference)
import jax
import jax.numpy as jnp
from jax.experimental import pallas as pl


def kernel(x, emb, bW0, bb0, bW1, bb1, bW2, bb2, tW0, tb0, tW1, tb1, tW2, tb2, tW3, tb3, tW4, tb4):
    raise NotImplementedError("write your pallas kernel here")


# speedup vs baseline: 10.6728x; 10.6728x over previous
import functools

import jax
import jax.numpy as jnp
import numpy as np
from jax import lax
from jax.experimental import pallas as pl
from jax.experimental.pallas import tpu as pltpu
from jax.experimental.pallas import tpu_sc as plsc

VOCAB = 1000000
EMBED = 128
NUM_DENSE = 13
NUM_SPARSE = 26

NC = 2
NS = 16
NW = NC * NS


def _sc_gather(table, idx):
    n = idx.shape[0]
    per_w = n // NW
    assert per_w % 128 == 0
    n_streams = per_w // 128
    group = 4
    rows_per_group = group * 128
    n_groups = n_streams // group
    assert n_groups * group == n_streams

    mesh = plsc.VectorSubcoreMesh(core_axis_name="c", subcore_axis_name="s")

    @functools.partial(
        pl.kernel,
        out_type=jax.ShapeDtypeStruct((n, EMBED), jnp.float32),
        mesh=mesh,
        scratch_types=[
            pltpu.VMEM((per_w,), jnp.int32),
            pltpu.VMEM((rows_per_group, EMBED), jnp.float32),
            pltpu.SemaphoreType.DMA,
        ],
    )
    def gather_kernel(table_hbm, idx_hbm, out_hbm, idx_v, rows_v, sem):
        wid = lax.axis_index("s") * NC + lax.axis_index("c")
        base = wid * per_w
        pltpu.sync_copy(idx_hbm.at[pl.ds(base, per_w)], idx_v)

        def gbody(g, carry):
            start = g * rows_per_group
            copies = []
            for j in range(group):
                copies.append(
                    pltpu.async_copy(
                        table_hbm.at[idx_v.at[pl.ds(start + j * 128, 128)]],
                        rows_v.at[pl.ds(j * 128, 128)],
                        sem,
                    )
                )
            for c in copies:
                c.wait()
            pltpu.sync_copy(rows_v, out_hbm.at[pl.ds(base + start, rows_per_group)])
            return carry

        lax.fori_loop(0, n_groups, gbody, 0)

    return gather_kernel(table, idx)


def _tc_forward(dense, gath, bW0, bb0, bW1, bb1, bW2, bb2,
                tW0h, tW0s, tb0, tW1, tb1, tW2, tb2, tW3, tb3, tW4, tb4):
    B = dense.shape[0]
    BB = 512
    grid = B // BB

    def body(dense_r, gath_r, bW0_r, bb0_r, bW1_r, bb1_r, bW2_r, bb2_r,
             tW0h_r, tW0s_r, tb0_r, tW1_r, tb1_r, tW2_r, tb2_r, tW3_r,
             tb3_r, tW4_r, tb4_r, out_r):
        d = dense_r[...]
        h = jnp.maximum(jnp.dot(d, bW0_r[...], preferred_element_type=jnp.float32) + bb0_r[...], 0.0)
        h = jnp.maximum(jnp.dot(h, bW1_r[...], preferred_element_type=jnp.float32) + bb1_r[...], 0.0)
        h = jnp.maximum(jnp.dot(h, bW2_r[...], preferred_element_type=jnp.float32) + bb2_r[...], 0.0)
        g = gath_r[...]
        c = jnp.concatenate([h[:, None, :], g], axis=1)
        inter = lax.dot_general(c, c, (((2,), (2,)), ((0,), (0,))),
                                preferred_element_type=jnp.float32)
        iflat = inter.reshape(BB, 27 * 27)
        z = (jnp.dot(h, tW0h_r[...], preferred_element_type=jnp.float32)
             + jnp.dot(iflat, tW0s_r[...], preferred_element_type=jnp.float32)
             + tb0_r[...])
        z = jnp.maximum(z, 0.0)
        z = jnp.maximum(jnp.dot(z, tW1_r[...], preferred_element_type=jnp.float32) + tb1_r[...], 0.0)
        z = jnp.maximum(jnp.dot(z, tW2_r[...], preferred_element_type=jnp.float32) + tb2_r[...], 0.0)
        z = jnp.maximum(jnp.dot(z, tW3_r[...], preferred_element_type=jnp.float32) + tb3_r[...], 0.0)
        out_r[...] = jnp.dot(z, tW4_r[...], preferred_element_type=jnp.float32) + tb4_r[...]

    def full(w):
        return pl.BlockSpec(w.shape, lambda i: (0,) * w.ndim)

    return pl.pallas_call(
        body,
        grid=(grid,),
        in_specs=[
            pl.BlockSpec((BB, NUM_DENSE), lambda i: (i, 0)),
            pl.BlockSpec((BB, NUM_SPARSE, EMBED), lambda i: (i, 0, 0)),
            full(bW0), full(bb0), full(bW1), full(bb1), full(bW2), full(bb2),
            full(tW0h), full(tW0s), full(tb0), full(tW1), full(tb1),
            full(tW2), full(tb2), full(tW3), full(tb3), full(tW4), full(tb4),
        ],
        out_specs=pl.BlockSpec((BB, 1), lambda i: (i, 0)),
        out_shape=jax.ShapeDtypeStruct((B, 1), jnp.float32),
    )(dense, gath, bW0, bb0, bW1, bb1, bW2, bb2,
      tW0h, tW0s, tb0, tW1, tb1, tW2, tb2, tW3, tb3, tW4, tb4)


_IU = np.triu_indices(NUM_SPARSE + 1)
_SCATTER_ROWS = np.asarray(_IU[0] * (NUM_SPARSE + 1) + _IU[1], dtype=np.int32)


def kernel(x, emb, bW0, bb0, bW1, bb1, bW2, bb2,
           tW0, tb0, tW1, tb1, tW2, tb2, tW3, tb3, tW4, tb4):
    B = x.shape[0]
    dense = x[:, :NUM_DENSE]
    idx = x[:, NUM_DENSE:].astype(jnp.int32).reshape(-1) % VOCAB

    gath = _sc_gather(emb, idx).reshape(B, NUM_SPARSE, EMBED)

    tW0h = tW0[:EMBED]
    tW0s = jnp.zeros((729, tW0.shape[1]), jnp.float32).at[_SCATTER_ROWS].set(tW0[EMBED:])

    return _tc_forward(dense, gath, bW0, bb0, bW1, bb1, bW2, bb2,
                       tW0h, tW0s, tb0, tW1, tb1, tW2, tb2, tW3, tb3, tW4, tb4)
